# Initial kernel scaffold; baseline (speedup 1.0000x reference)
import jax
import jax.numpy as jnp
from jax.experimental import pallas as pl


def kernel(x, edge_index, W1, b1, W2, b2, Wc1, bc1, Wc2, bc2):
    raise NotImplementedError("write your pallas kernel here")


# speedup vs baseline: 8.5477x; 8.5477x over previous
import functools

import jax
import jax.numpy as jnp
from jax.experimental import pallas as pl
from jax.experimental.pallas import tpu as pltpu
from jax.experimental.pallas import tpu_sc as plsc

N = 10000
E = 320000
F = 128
NUM_PROTO = 4
NC, NS = 2, 16
NW = NC * NS
CSZ = 64
CH = 160
EP = NW * CH * CSZ
DUMP = N
NPAD = 10240
RPS = NPAD // NS

_mesh = plsc.VectorSubcoreMesh(
    core_axis_name="c", subcore_axis_name="s", num_cores=NC, num_subcores=NS)


def _degree(idxw):
    @functools.partial(
        pl.kernel,
        out_type=jax.ShapeDtypeStruct((NC, NPAD, F), jnp.float32),
        mesh=_mesh,
        scratch_types=[
            pltpu.VMEM((2, CSZ), jnp.int32),
            pltpu.VMEM((CSZ, F), jnp.float32),
            pltpu.VMEM_SHARED((NPAD, F), jnp.float32),
        ],
    )
    def k(idx_hbm, out_hbm, idxv, ones_v, acc):
        c = jax.lax.axis_index("c")
        s = jax.lax.axis_index("s")
        w = c * NS + s

        @pl.loop(0, CSZ)
        def _(r):
            @pl.loop(0, F, step=16)
            def _(l):
                ones_v[r, pl.ds(l, 16)] = jnp.zeros((16,), jnp.float32)

        @pl.loop(0, RPS, step=CSZ)
        def _(r):
            pltpu.sync_copy(ones_v, acc.at[pl.ds(s * RPS + r, CSZ)])

        @pl.loop(0, CSZ)
        def _(r):
            @pl.loop(0, F, step=16)
            def _(l):
                ones_v[r, pl.ds(l, 16)] = jnp.full((16,), 1.0, jnp.float32)

        plsc.subcore_barrier()

        @pl.loop(0, CH)
        def _(g):
            pltpu.sync_copy(idx_hbm.at[w, g], idxv)
            pltpu.sync_copy(ones_v, acc.at[idxv.at[1]], add=True)

        plsc.subcore_barrier()

        @pl.loop(0, RPS, step=CSZ)
        def _(r):
            base = s * RPS + r
            pltpu.sync_copy(acc.at[pl.ds(base, CSZ)],
                            out_hbm.at[c, pl.ds(base, CSZ)])

    return k(idxw)


def _edge_sum(hs, idxw):
    @functools.partial(
        pl.kernel,
        out_type=jax.ShapeDtypeStruct((NC, NPAD, F), jnp.float32),
        mesh=_mesh,
        scratch_types=[
            pltpu.VMEM((2, CSZ), jnp.int32),
            pltpu.VMEM((2, CSZ), jnp.int32),
            pltpu.VMEM((CSZ, F), jnp.float32),
            pltpu.VMEM((CSZ, F), jnp.float32),
            pltpu.VMEM_SHARED((NPAD, F), jnp.float32),
            pltpu.SemaphoreType.DMA,
            pltpu.SemaphoreType.DMA,
        ],
    )
    def k(hs_hbm, idx_hbm, out_hbm, idxa, idxb, bufa, bufb, acc, sema, semb):
        c = jax.lax.axis_index("c")
        s = jax.lax.axis_index("s")
        w = c * NS + s

        @pl.loop(0, CSZ)
        def _(r):
            @pl.loop(0, F, step=16)
            def _(l):
                bufa[r, pl.ds(l, 16)] = jnp.zeros((16,), jnp.float32)

        @pl.loop(0, RPS, step=CSZ)
        def _(r):
            pltpu.sync_copy(bufa, acc.at[pl.ds(s * RPS + r, CSZ)])

        pltpu.sync_copy(idx_hbm.at[w, 0], idxa)
        plsc.subcore_barrier()

        pltpu.async_copy(hs_hbm.at[idxa.at[0]], bufa, sema)

        @pl.loop(0, CH, step=2)
        def _(g):
            pltpu.sync_copy(idx_hbm.at[w, g + 1], idxb)
            pltpu.make_async_copy(hs_hbm.at[idxa.at[0]], bufa, sema).wait()
            pltpu.async_copy(hs_hbm.at[idxb.at[0]], bufb, semb)
            pltpu.sync_copy(bufa, acc.at[idxa.at[1]], add=True)

            @pl.when(g + 2 < CH)
            def _():
                pltpu.sync_copy(idx_hbm.at[w, g + 2], idxa)

            pltpu.make_async_copy(hs_hbm.at[idxb.at[0]], bufb, semb).wait()

            @pl.when(g + 2 < CH)
            def _():
                pltpu.async_copy(hs_hbm.at[idxa.at[0]], bufa, sema)

            pltpu.sync_copy(bufb, acc.at[idxb.at[1]], add=True)

        plsc.subcore_barrier()

        @pl.loop(0, RPS, step=CSZ)
        def _(r):
            base = s * RPS + r
            pltpu.sync_copy(acc.at[pl.ds(base, CSZ)],
                            out_hbm.at[c, pl.ds(base, CSZ)])

    return k(hs, idxw)


def _dot(a, b):
    return jax.lax.dot_general(
        a, b, (((1,), (0,)), ((), ())),
        precision=jax.lax.Precision.HIGHEST,
        preferred_element_type=jnp.float32)


def _matmul(x, W):
    def body(x_ref, w_ref, o_ref):
        o_ref[...] = _dot(x_ref[...], w_ref[...])

    return pl.pallas_call(
        body,
        out_shape=jax.ShapeDtypeStruct((x.shape[0], W.shape[1]), jnp.float32),
    )(x, W)


def _scale_rows(h, d0, d1):
    def body(h_ref, d0_ref, d1_ref, o_ref):
        deg = 1.0 + d0_ref[:, :1] + d1_ref[:, :1]
        o_ref[...] = h_ref[...] * jax.lax.rsqrt(deg)

    return pl.pallas_call(
        body,
        out_shape=jax.ShapeDtypeStruct(h.shape, jnp.float32),
    )(h, d0, d1)


def _layer_mid(s0, s1, hs, d0, d1, b, W):
    def body(s0_ref, s1_ref, hs_ref, d0_ref, d1_ref, b_ref, w_ref, o_ref):
        deg = 1.0 + d0_ref[:, :1] + d1_ref[:, :1]
        dis = jax.lax.rsqrt(deg)
        t = dis * (s0_ref[...] + s1_ref[...] + hs_ref[...]) + b_ref[...]
        t = jnp.maximum(t, 0.0)
        o_ref[...] = dis * _dot(t, w_ref[...])

    return pl.pallas_call(
        body,
        out_shape=jax.ShapeDtypeStruct(hs.shape, jnp.float32),
    )(s0, s1, hs, d0, d1, b, W)


def _layer_last(s0, s1, hs, d0, d1, b):
    def body(s0_ref, s1_ref, hs_ref, d0_ref, d1_ref, b_ref, o_ref):
        deg = 1.0 + d0_ref[:, :1] + d1_ref[:, :1]
        dis = jax.lax.rsqrt(deg)
        t = dis * (s0_ref[...] + s1_ref[...] + hs_ref[...]) + b_ref[...]
        o_ref[...] = jnp.maximum(t, 0.0)

    return pl.pallas_call(
        body,
        out_shape=jax.ShapeDtypeStruct(hs.shape, jnp.float32),
    )(s0, s1, hs, d0, d1, b)


def _classifier(xr, Wc1, bc1, Wc2, bc2):
    def body(x_ref, w1_ref, b1_ref, w2_ref, b2_ref, o_ref):
        h = jnp.maximum(_dot(x_ref[...], w1_ref[...]) + b1_ref[...], 0.0)
        logits = _dot(h, w2_ref[...]) + b2_ref[...]
        m = jnp.max(logits, axis=1, keepdims=True)
        z = logits - m
        lse = jnp.log(jnp.sum(jnp.exp(z), axis=1, keepdims=True))
        o_ref[...] = z - lse

    n = xr.shape[0]
    return pl.pallas_call(
        body,
        out_shape=jax.ShapeDtypeStruct((n, Wc2.shape[1]), jnp.float32),
    )(xr, Wc1, bc1, Wc2, bc2)


def kernel(x, edge_index, W1, b1, W2, b2, Wc1, bc1, Wc2, bc2):
    e32 = edge_index.astype(jnp.int32)
    pad = EP - E
    srcw = jnp.concatenate(
        [e32[0], jnp.zeros((pad,), jnp.int32)]).reshape(NW, CH, CSZ)
    dstw = jnp.concatenate(
        [e32[1], jnp.full((pad,), DUMP, jnp.int32)]).reshape(NW, CH, CSZ)
    idxw = jnp.stack([srcw, dstw], axis=2)

    b1r = b1.reshape(1, F)
    b2r = b2.reshape(1, F)
    bc1r = bc1.reshape(1, -1)
    bc2r = bc2.reshape(1, -1)

    degp = _degree(idxw)
    d0 = degp[0, :N]
    d1 = degp[1, :N]
    h1 = _matmul(x, W1)

    hs1 = _scale_rows(h1, d0, d1)
    s1 = _edge_sum(hs1, idxw)
    hs2 = _layer_mid(s1[0, :N], s1[1, :N], hs1, d0, d1, b1r, W2)
    s2 = _edge_sum(hs2, idxw)
    g2 = _layer_last(s2[0, :N], s2[1, :N], hs2, d0, d1, b2r)

    xr = g2.reshape(N // NUM_PROTO, F * NUM_PROTO)
    return _classifier(xr, Wc1, bc1r, Wc2, bc2r)


# speedup vs baseline: 9.5305x; 1.1150x over previous
import functools

import jax
import jax.numpy as jnp
from jax.experimental import pallas as pl
from jax.experimental.pallas import tpu as pltpu
from jax.experimental.pallas import tpu_sc as plsc

N = 10000
E = 320000
F = 128
NUM_PROTO = 4
NC, NS = 2, 16
NW = NC * NS
CSZ = 128
CH = 80
EP = NW * CH * CSZ
EPW = E // NW
PADW = CH * CSZ - EPW
NPAD = 10240
RPS = NPAD // NS

_mesh = plsc.VectorSubcoreMesh(
    core_axis_name="c", subcore_axis_name="s", num_cores=NC, num_subcores=NS)


def _degree(idxw):
    @functools.partial(
        pl.kernel,
        out_type=jax.ShapeDtypeStruct((NC, NPAD, F), jnp.float32),
        mesh=_mesh,
        scratch_types=[
            pltpu.VMEM((CH, 2, CSZ), jnp.int32),
            pltpu.VMEM((CSZ, F), jnp.float32),
            pltpu.VMEM_SHARED((NPAD, F), jnp.float32),
            pltpu.SemaphoreType.DMA,
        ],
    )
    def k(idx_hbm, out_hbm, idxv, ones_v, acc, sem):
        c = jax.lax.axis_index("c")
        s = jax.lax.axis_index("s")
        w = c * NS + s

        @pl.loop(0, CSZ)
        def _(r):
            @pl.loop(0, F, step=16)
            def _(l):
                ones_v[r, pl.ds(l, 16)] = jnp.zeros((16,), jnp.float32)

        @pl.loop(0, RPS, step=CSZ)
        def _(r):
            pltpu.sync_copy(ones_v, acc.at[pl.ds(s * RPS + r, CSZ)])

        @pl.loop(0, CSZ)
        def _(r):
            @pl.loop(0, F, step=16)
            def _(l):
                ones_v[r, pl.ds(l, 16)] = jnp.full((16,), 1.0, jnp.float32)

        pltpu.sync_copy(idx_hbm.at[w], idxv)
        plsc.subcore_barrier()

        @pl.loop(0, CH, step=4)
        def _(g):
            for j in range(4):
                pltpu.async_copy(ones_v, acc.at[idxv.at[g + j, 1]], sem,
                                 add=True)
            for j in range(4):
                pltpu.make_async_copy(
                    ones_v, acc.at[idxv.at[g + j, 1]], sem).wait()

        plsc.subcore_barrier()

        @pl.loop(0, RPS, step=CSZ)
        def _(r):
            base = s * RPS + r
            pltpu.sync_copy(acc.at[pl.ds(base, CSZ)],
                            out_hbm.at[c, pl.ds(base, CSZ)])

    return k(idxw)


def _edge_sum(hs, idxw):
    @functools.partial(
        pl.kernel,
        out_type=jax.ShapeDtypeStruct((NC, NPAD, F), jnp.float32),
        mesh=_mesh,
        scratch_types=[
            pltpu.VMEM((2, 2, CSZ), jnp.int32),
            pltpu.VMEM((2, 2, CSZ), jnp.int32),
            pltpu.VMEM((CSZ, F), jnp.float32),
            pltpu.VMEM((CSZ, F), jnp.float32),
            pltpu.VMEM_SHARED((NPAD, F), jnp.float32),
            pltpu.SemaphoreType.DMA,
            pltpu.SemaphoreType.DMA,
            pltpu.SemaphoreType.DMA,
        ],
    )
    def k(hs_hbm, idx_hbm, out_hbm, ia, ib, bufa, bufb, acc,
          sema, semb, semi):
        c = jax.lax.axis_index("c")
        s = jax.lax.axis_index("s")
        w = c * NS + s

        @pl.loop(0, CSZ)
        def _(r):
            @pl.loop(0, F, step=16)
            def _(l):
                bufa[r, pl.ds(l, 16)] = jnp.zeros((16,), jnp.float32)

        @pl.loop(0, RPS, step=CSZ)
        def _(r):
            pltpu.sync_copy(bufa, acc.at[pl.ds(s * RPS + r, CSZ)])

        pltpu.sync_copy(idx_hbm.at[w, pl.ds(0, 2)], ia)
        plsc.subcore_barrier()

        pltpu.async_copy(hs_hbm.at[ia.at[0, 0]], bufa, sema)
        pltpu.async_copy(hs_hbm.at[ia.at[1, 0]], bufb, semb)

        @pl.loop(0, CH, step=4)
        def _(g):
            @pl.when(g + 2 < CH)
            def _():
                pltpu.async_copy(idx_hbm.at[w, pl.ds(g + 2, 2)], ib, semi)

            pltpu.make_async_copy(hs_hbm.at[ia.at[0, 0]], bufa, sema).wait()
            pltpu.sync_copy(bufa, acc.at[ia.at[0, 1]], add=True)
            pltpu.make_async_copy(hs_hbm.at[ia.at[1, 0]], bufb, semb).wait()
            pltpu.sync_copy(bufb, acc.at[ia.at[1, 1]], add=True)

            @pl.when(g + 2 < CH)
            def _():
                pltpu.make_async_copy(
                    idx_hbm.at[w, pl.ds(g + 2, 2)], ib, semi).wait()
                pltpu.async_copy(hs_hbm.at[ib.at[0, 0]], bufa, sema)
                pltpu.async_copy(hs_hbm.at[ib.at[1, 0]], bufb, semb)

            @pl.when(g + 4 < CH)
            def _():
                pltpu.async_copy(idx_hbm.at[w, pl.ds(g + 4, 2)], ia, semi)

            @pl.when(g + 2 < CH)
            def _():
                pltpu.make_async_copy(
                    hs_hbm.at[ib.at[0, 0]], bufa, sema).wait()
                pltpu.sync_copy(bufa, acc.at[ib.at[0, 1]], add=True)
                pltpu.make_async_copy(
                    hs_hbm.at[ib.at[1, 0]], bufb, semb).wait()
                pltpu.sync_copy(bufb, acc.at[ib.at[1, 1]], add=True)

            @pl.when(g + 4 < CH)
            def _():
                pltpu.make_async_copy(
                    idx_hbm.at[w, pl.ds(g + 4, 2)], ia, semi).wait()
                pltpu.async_copy(hs_hbm.at[ia.at[0, 0]], bufa, sema)
                pltpu.async_copy(hs_hbm.at[ia.at[1, 0]], bufb, semb)

        plsc.subcore_barrier()

        @pl.loop(0, RPS, step=CSZ)
        def _(r):
            base = s * RPS + r
            pltpu.sync_copy(acc.at[pl.ds(base, CSZ)],
                            out_hbm.at[c, pl.ds(base, CSZ)])

    return k(hs, idxw)


def _dot(a, b):
    return jax.lax.dot_general(
        a, b, (((1,), (0,)), ((), ())),
        precision=jax.lax.Precision.HIGHEST,
        preferred_element_type=jnp.float32)


def _matmul(x, W):
    def body(x_ref, w_ref, o_ref):
        o_ref[...] = _dot(x_ref[...], w_ref[...])

    return pl.pallas_call(
        body,
        out_shape=jax.ShapeDtypeStruct((x.shape[0], W.shape[1]), jnp.float32),
    )(x, W)


def _scale_rows(h, d0, d1):
    def body(h_ref, d0_ref, d1_ref, o_ref):
        deg = 1.0 + d0_ref[:, :1] + d1_ref[:, :1]
        o_ref[...] = h_ref[...] * jax.lax.rsqrt(deg)

    return pl.pallas_call(
        body,
        out_shape=jax.ShapeDtypeStruct(h.shape, jnp.float32),
    )(h, d0, d1)


def _layer_mid(s0, s1, hs, d0, d1, b, W):
    def body(s0_ref, s1_ref, hs_ref, d0_ref, d1_ref, b_ref, w_ref, o_ref):
        deg = 1.0 + d0_ref[:, :1] + d1_ref[:, :1]
        dis = jax.lax.rsqrt(deg)
        t = dis * (s0_ref[...] + s1_ref[...] + hs_ref[...]) + b_ref[...]
        t = jnp.maximum(t, 0.0)
        o_ref[...] = dis * _dot(t, w_ref[...])

    return pl.pallas_call(
        body,
        out_shape=jax.ShapeDtypeStruct(hs.shape, jnp.float32),
    )(s0, s1, hs, d0, d1, b, W)


def _layer_last(s0, s1, hs, d0, d1, b):
    def body(s0_ref, s1_ref, hs_ref, d0_ref, d1_ref, b_ref, o_ref):
        deg = 1.0 + d0_ref[:, :1] + d1_ref[:, :1]
        dis = jax.lax.rsqrt(deg)
        t = dis * (s0_ref[...] + s1_ref[...] + hs_ref[...]) + b_ref[...]
        o_ref[...] = jnp.maximum(t, 0.0)

    return pl.pallas_call(
        body,
        out_shape=jax.ShapeDtypeStruct(hs.shape, jnp.float32),
    )(s0, s1, hs, d0, d1, b)


def _classifier(xr, Wc1, bc1, Wc2, bc2):
    def body(x_ref, w1_ref, b1_ref, w2_ref, b2_ref, o_ref):
        h = jnp.maximum(_dot(x_ref[...], w1_ref[...]) + b1_ref[...], 0.0)
        logits = _dot(h, w2_ref[...]) + b2_ref[...]
        m = jnp.max(logits, axis=1, keepdims=True)
        z = logits - m
        lse = jnp.log(jnp.sum(jnp.exp(z), axis=1, keepdims=True))
        o_ref[...] = z - lse

    n = xr.shape[0]
    return pl.pallas_call(
        body,
        out_shape=jax.ShapeDtypeStruct((n, Wc2.shape[1]), jnp.float32),
    )(xr, Wc1, bc1, Wc2, bc2)


def kernel(x, edge_index, W1, b1, W2, b2, Wc1, bc1, Wc2, bc2):
    e32 = edge_index.astype(jnp.int32)
    pad_src = jnp.zeros((NW, PADW), jnp.int32)
    pad_dst = jnp.broadcast_to(jnp.arange(PADW, dtype=jnp.int32) + N,
                               (NW, PADW))
    srcw = jnp.concatenate([e32[0].reshape(NW, EPW), pad_src], axis=1)
    dstw = jnp.concatenate([e32[1].reshape(NW, EPW), pad_dst], axis=1)
    idxw = jnp.stack([srcw.reshape(NW, CH, CSZ),
                      dstw.reshape(NW, CH, CSZ)], axis=2)

    b1r = b1.reshape(1, F)
    b2r = b2.reshape(1, F)
    bc1r = bc1.reshape(1, -1)
    bc2r = bc2.reshape(1, -1)

    degp = _degree(idxw)
    d0 = degp[0, :N]
    d1 = degp[1, :N]
    h1 = _matmul(x, W1)

    hs1 = _scale_rows(h1, d0, d1)
    s1 = _edge_sum(hs1, idxw)
    hs2 = _layer_mid(s1[0, :N], s1[1, :N], hs1, d0, d1, b1r, W2)
    s2 = _edge_sum(hs2, idxw)
    g2 = _layer_last(s2[0, :N], s2[1, :N], hs2, d0, d1, b2r)

    xr = g2.reshape(N // NUM_PROTO, F * NUM_PROTO)
    return _classifier(xr, Wc1, bc1r, Wc2, bc2r)


# speedup vs baseline: 13.6568x; 1.4330x over previous
import functools

import jax
import jax.numpy as jnp
from jax.experimental import pallas as pl
from jax.experimental.pallas import tpu as pltpu
from jax.experimental.pallas import tpu_sc as plsc

N = 10000
E = 320000
F = 128
FH = F // 2
NUM_PROTO = 4
NC, NS = 2, 16
CSZ = 64
CH = 320
CHD = CH // NC
EP = NS * CH * CSZ
EPW = E // NS
PADW = CH * CSZ - EPW
NDUMP = 240
NPAD = 10240
RPS = NPAD // NS

_mesh = plsc.VectorSubcoreMesh(
    core_axis_name="c", subcore_axis_name="s", num_cores=NC, num_subcores=NS)


def _degree(idxs):
    @functools.partial(
        pl.kernel,
        out_type=jax.ShapeDtypeStruct((NC, NPAD, F), jnp.float32),
        mesh=_mesh,
        scratch_types=[
            pltpu.VMEM((CHD, 2, CSZ), jnp.int32),
            pltpu.VMEM((CSZ, F), jnp.float32),
            pltpu.VMEM_SHARED((NPAD, F), jnp.float32),
            pltpu.SemaphoreType.DMA,
        ],
    )
    def k(idx_hbm, out_hbm, idxv, ones_v, acc, sem):
        c = jax.lax.axis_index("c")
        s = jax.lax.axis_index("s")

        @pl.loop(0, CSZ)
        def _(r):
            @pl.loop(0, F, step=16)
            def _(l):
                ones_v[r, pl.ds(l, 16)] = jnp.zeros((16,), jnp.float32)

        @pl.loop(0, RPS, step=CSZ)
        def _(r):
            pltpu.sync_copy(ones_v, acc.at[pl.ds(s * RPS + r, CSZ)])

        @pl.loop(0, CSZ)
        def _(r):
            @pl.loop(0, F, step=16)
            def _(l):
                ones_v[r, pl.ds(l, 16)] = jnp.full((16,), 1.0, jnp.float32)

        pltpu.sync_copy(idx_hbm.at[s, pl.ds(c * CHD, CHD)], idxv)
        plsc.subcore_barrier()

        @pl.loop(0, CHD, step=4)
        def _(g):
            for j in range(4):
                pltpu.async_copy(ones_v, acc.at[idxv.at[g + j, 1]], sem,
                                 add=True)
            for j in range(4):
                pltpu.make_async_copy(
                    ones_v, acc.at[idxv.at[g + j, 1]], sem).wait()

        plsc.subcore_barrier()
        pltpu.sync_copy(acc.at[pl.ds(s * RPS, RPS)],
                        out_hbm.at[c, pl.ds(s * RPS, RPS)])

    return k(idxs)


NBUF = 8


def _edge_sum(hs_a, hs_b, idxs):
    @functools.partial(
        pl.kernel,
        out_type=jax.ShapeDtypeStruct((NC, NPAD, FH), jnp.float32),
        mesh=_mesh,
        scratch_types=(
            [pltpu.VMEM((NBUF, 2, CSZ), jnp.int32)] * 2 +
            [pltpu.VMEM((CSZ, FH), jnp.float32)] * NBUF +
            [pltpu.VMEM_SHARED((NPAD, FH), jnp.float32)] +
            [pltpu.SemaphoreType.DMA] * NBUF +
            [pltpu.SemaphoreType.DMA]
        ),
        compiler_params=pltpu.CompilerParams(use_tc_tiling_on_sc=False),
    )
    def k(hsa_hbm, hsb_hbm, idx_hbm, out_hbm, ia, ib, *rest):
        bufs = rest[:NBUF]
        acc = rest[NBUF]
        sems = rest[NBUF + 1:2 * NBUF + 1]
        semi = rest[2 * NBUF + 1]
        c = jax.lax.axis_index("c")
        s = jax.lax.axis_index("s")

        b0 = bufs[0]

        @pl.loop(0, CSZ)
        def _(r):
            @pl.loop(0, FH, step=16)
            def _(l):
                b0[r, pl.ds(l, 16)] = jnp.zeros((16,), jnp.float32)

        @pl.loop(0, RPS, step=CSZ)
        def _(r):
            pltpu.sync_copy(b0, acc.at[pl.ds(s * RPS + r, CSZ)])

        pltpu.sync_copy(idx_hbm.at[s, pl.ds(0, NBUF)], ia)
        pltpu.async_copy(idx_hbm.at[s, pl.ds(NBUF, NBUF)], ib, semi)
        plsc.subcore_barrier()

        def pipeline(table):
            for j in range(NBUF):
                pltpu.async_copy(table.at[ia.at[j, 0]], bufs[j], sems[j])

            def block(G, icur, inxt):
                for j in range(NBUF):
                    pltpu.make_async_copy(
                        table.at[icur.at[j, 0]], bufs[j], sems[j]).wait()
                    pltpu.async_copy(bufs[j], acc.at[icur.at[j, 1]],
                                     sems[j], add=True)

                @pl.when(G + NBUF < CH)
                def _():
                    pltpu.make_async_copy(
                        idx_hbm.at[s, pl.ds(G + NBUF, NBUF)], inxt,
                        semi).wait()
                    for j in range(NBUF):
                        pltpu.make_async_copy(
                            bufs[j], acc.at[icur.at[j, 1]], sems[j]).wait()
                        pltpu.async_copy(table.at[inxt.at[j, 0]], bufs[j],
                                         sems[j])

                    @pl.when(G + 2 * NBUF < CH)
                    def _():
                        pltpu.async_copy(
                            idx_hbm.at[s, pl.ds(G + 2 * NBUF, NBUF)], icur,
                            semi)

            @pl.loop(0, CH, step=2 * NBUF)
            def _(g):
                block(g, ia, ib)
                block(g + NBUF, ib, ia)

            for j in range(NBUF):
                pltpu.make_async_copy(bufs[j], acc.at[ib.at[j, 1]],
                                      sems[j]).wait()

        @pl.when(c == 0)
        def _():
            pipeline(hsa_hbm)

        @pl.when(c == 1)
        def _():
            pipeline(hsb_hbm)

        plsc.subcore_barrier()
        pltpu.sync_copy(acc.at[pl.ds(s * RPS, RPS)],
                        out_hbm.at[c, pl.ds(s * RPS, RPS)])

    return k(hs_a, hs_b, idxs)


def _dot(a, b):
    return jax.lax.dot_general(
        a, b, (((1,), (0,)), ((), ())),
        precision=jax.lax.Precision.HIGHEST,
        preferred_element_type=jnp.float32)


def _unsplit(ref3, rows):
    return jnp.concatenate([ref3[0, :rows], ref3[1, :rows]], axis=1)


def _dis(degp):
    def body(degp_ref, o_ref):
        deg = 1.0 + degp_ref[0, :N, :1] + degp_ref[1, :N, :1]
        o_ref[...] = jnp.broadcast_to(jax.lax.rsqrt(deg), (N, 8))

    return pl.pallas_call(
        body,
        out_shape=jax.ShapeDtypeStruct((N, 8), jnp.float32),
    )(degp)


BR = 2000


def _matmul_scale(x, W, dis):
    def body(x_ref, w_ref, dis_ref, o_ref):
        h = _dot(x_ref[...], w_ref[...]) * dis_ref[:, :1]
        o_ref[0] = h[:, :FH]
        o_ref[1] = h[:, FH:]

    return pl.pallas_call(
        body,
        grid=(N // BR,),
        in_specs=[
            pl.BlockSpec((BR, F), lambda i: (i, 0)),
            pl.BlockSpec((F, F), lambda i: (0, 0)),
            pl.BlockSpec((BR, 8), lambda i: (i, 0)),
        ],
        out_specs=pl.BlockSpec((NC, BR, FH), lambda i: (0, i, 0)),
        out_shape=jax.ShapeDtypeStruct((NC, N, FH), jnp.float32),
    )(x, W, dis)


def _layer_mid(sp, hs, dis_in, b, W):
    def body(sp_ref, hs_ref, dis_ref, b_ref, w_ref, o_ref):
        dis = dis_ref[:, :1]
        t = dis * (_unsplit(sp_ref, BR) + _unsplit(hs_ref, BR)) + b_ref[...]
        t = jnp.maximum(t, 0.0)
        h = dis * _dot(t, w_ref[...])
        o_ref[0] = h[:, :FH]
        o_ref[1] = h[:, FH:]

    return pl.pallas_call(
        body,
        grid=(N // BR,),
        in_specs=[
            pl.BlockSpec((NC, BR, FH), lambda i: (0, i, 0)),
            pl.BlockSpec((NC, BR, FH), lambda i: (0, i, 0)),
            pl.BlockSpec((BR, 8), lambda i: (i, 0)),
            pl.BlockSpec((1, F), lambda i: (0, 0)),
            pl.BlockSpec((F, F), lambda i: (0, 0)),
        ],
        out_specs=pl.BlockSpec((NC, BR, FH), lambda i: (0, i, 0)),
        out_shape=jax.ShapeDtypeStruct((NC, N, FH), jnp.float32),
    )(sp, hs, dis_in, b, W)


def _layer_last(sp, hs, dis_in, b):
    def body(sp_ref, hs_ref, dis_ref, b_ref, o_ref):
        dis = dis_ref[:, :1]
        t = dis * (_unsplit(sp_ref, BR) + _unsplit(hs_ref, BR)) + b_ref[...]
        o_ref[...] = jnp.maximum(t, 0.0)

    return pl.pallas_call(
        body,
        grid=(N // BR,),
        in_specs=[
            pl.BlockSpec((NC, BR, FH), lambda i: (0, i, 0)),
            pl.BlockSpec((NC, BR, FH), lambda i: (0, i, 0)),
            pl.BlockSpec((BR, 8), lambda i: (i, 0)),
            pl.BlockSpec((1, F), lambda i: (0, 0)),
        ],
        out_specs=pl.BlockSpec((BR, F), lambda i: (i, 0)),
        out_shape=jax.ShapeDtypeStruct((N, F), jnp.float32),
    )(sp, hs, dis_in, b)


def _classifier(xr, Wc1, bc1, Wc2, bc2):
    def body(x_ref, w1_ref, b1_ref, w2_ref, b2_ref, o_ref):
        h = jnp.maximum(_dot(x_ref[...], w1_ref[...]) + b1_ref[...], 0.0)
        logits = _dot(h, w2_ref[...]) + b2_ref[...]
        m = jnp.max(logits, axis=1, keepdims=True)
        z = logits - m
        lse = jnp.log(jnp.sum(jnp.exp(z), axis=1, keepdims=True))
        o_ref[...] = z - lse

    n = xr.shape[0]
    return pl.pallas_call(
        body,
        out_shape=jax.ShapeDtypeStruct((n, Wc2.shape[1]), jnp.float32),
    )(xr, Wc1, bc1, Wc2, bc2)


def kernel(x, edge_index, W1, b1, W2, b2, Wc1, bc1, Wc2, bc2):
    e32 = edge_index.astype(jnp.int32)
    pad_src = jnp.zeros((NS, PADW), jnp.int32)
    pad_dst = jnp.broadcast_to(
        jnp.arange(PADW, dtype=jnp.int32) % NDUMP + N, (NS, PADW))
    srcw = jnp.concatenate([e32[0].reshape(NS, EPW), pad_src], axis=1)
    dstw = jnp.concatenate([e32[1].reshape(NS, EPW), pad_dst], axis=1)
    idxs = jnp.stack([srcw.reshape(NS, CH, CSZ),
                      dstw.reshape(NS, CH, CSZ)], axis=2)

    b1r = b1.reshape(1, F)
    b2r = b2.reshape(1, F)
    bc1r = bc1.reshape(1, -1)
    bc2r = bc2.reshape(1, -1)

    degp = _degree(idxs)
    dis = _dis(degp)
    hs1 = _matmul_scale(x, W1, dis)
    s1 = _edge_sum(hs1[0], hs1[1], idxs)
    hs2 = _layer_mid(s1, hs1, dis, b1r, W2)
    s2 = _edge_sum(hs2[0], hs2[1], idxs)
    g2 = _layer_last(s2, hs2, dis, b2r)

    xr = g2.reshape(N // NUM_PROTO, F * NUM_PROTO)
    return _classifier(xr, Wc1, bc1r, Wc2, bc2r)
